# Initial kernel scaffold; baseline (speedup 1.0000x reference)
import jax
import jax.numpy as jnp
from jax.experimental import pallas as pl


def kernel(x, in_proj_w, conv_w, conv_b, x_proj_w, dt_proj_w, dt_proj_b, A_log, D, out_proj_w):
    raise NotImplementedError("write your pallas kernel here")


# speedup vs baseline: 18.0562x; 18.0562x over previous
import jax
import jax.numpy as jnp
from jax.experimental import pallas as pl
from jax.experimental.pallas import tpu as pltpu

B_, L, DM = 2, 1024, 1024
DI, DS, DC, DR = 2048, 16, 4, 64
F32, BF16 = jnp.float32, jnp.bfloat16

E1 = 512
LB2 = 512
LC = 256
EG = 512
LB4 = 256


def _silu(x):
    return x * jax.nn.sigmoid(x)


def _softplus(x):
    return jnp.maximum(x, 0.0) + jnp.log(1.0 + jnp.exp(-jnp.abs(x)))


def _k1_body(xb_ref, wit_ref, wzt_ref, cwt_ref, cb_ref, xc_ref, zs_ref):
    xb = xb_ref[0]
    xi = jnp.dot(xb, wit_ref[...], preferred_element_type=F32)
    z = jnp.dot(xb, wzt_ref[...], preferred_element_type=F32)
    acc = cwt_ref[3:4, :] * xi + cb_ref[...]
    for k in range(3):
        sh = 3 - k
        xs = jnp.concatenate(
            [jnp.zeros((sh, E1), F32), xi[:L - sh, :]], axis=0)
        acc = acc + cwt_ref[k:k + 1, :] * xs
    xc_ref[0] = _silu(acc)
    zs_ref[0] = _silu(z).astype(BF16)


def _k1(xb, wit, wzt, cwt, cb):
    ne = DI // E1
    return pl.pallas_call(
        _k1_body,
        grid=(B_, ne),
        in_specs=[
            pl.BlockSpec((1, L, DM), lambda b, e: (b, 0, 0)),
            pl.BlockSpec((DM, E1), lambda b, e: (0, e)),
            pl.BlockSpec((DM, E1), lambda b, e: (0, e)),
            pl.BlockSpec((DC, E1), lambda b, e: (0, e)),
            pl.BlockSpec((1, E1), lambda b, e: (0, e)),
        ],
        out_specs=[
            pl.BlockSpec((1, L, E1), lambda b, e: (b, 0, e)),
            pl.BlockSpec((1, L, E1), lambda b, e: (b, 0, e)),
        ],
        out_shape=[
            jax.ShapeDtypeStruct((B_, L, DI), F32),
            jax.ShapeDtypeStruct((B_, L, DI), BF16),
        ],
        compiler_params=pltpu.CompilerParams(
            dimension_semantics=("parallel", "arbitrary"),
        ),
        name="mamba_inproj_conv",
    )(xb, wit, wzt, cwt, cb)


def _k2_body(xc_ref, xpwt_ref, dtwt_ref, dtb_ref, dt_ref, b_ref, c_ref):
    xcb = xc_ref[0].astype(BF16)
    dbl = jnp.dot(xcb, xpwt_ref[...], preferred_element_type=F32)
    dti = dbl[:, :DR].astype(BF16)
    pre = jnp.dot(dti, dtwt_ref[...], preferred_element_type=F32) + dtb_ref[...]
    dt_ref[0] = _softplus(pre)
    b_ref[0] = dbl[:, DR:DR + DS].astype(BF16)
    c_ref[0] = dbl[:, DR + DS:DR + 2 * DS].astype(BF16)


def _k2(xc, xpwt, dtwt, dtb):
    nl = L // LB2
    return pl.pallas_call(
        _k2_body,
        grid=(B_, nl),
        in_specs=[
            pl.BlockSpec((1, LB2, DI), lambda b, l: (b, l, 0)),
            pl.BlockSpec((DI, DR + 2 * DS), lambda b, l: (0, 0)),
            pl.BlockSpec((DR, DI), lambda b, l: (0, 0)),
            pl.BlockSpec((1, DI), lambda b, l: (0, 0)),
        ],
        out_specs=[
            pl.BlockSpec((1, LB2, DI), lambda b, l: (b, l, 0)),
            pl.BlockSpec((1, LB2, DS), lambda b, l: (b, l, 0)),
            pl.BlockSpec((1, LB2, DS), lambda b, l: (b, l, 0)),
        ],
        out_shape=[
            jax.ShapeDtypeStruct((B_, L, DI), F32),
            jax.ShapeDtypeStruct((B_, L, DS), BF16),
            jax.ShapeDtypeStruct((B_, L, DS), BF16),
        ],
        compiler_params=pltpu.CompilerParams(
            dimension_semantics=("parallel", "arbitrary"),
        ),
        name="mamba_xproj_dt",
    )(xc, xpwt, dtwt, dtb)


def _k3_body(dt_ref, xc_ref, zs_ref, bb_ref, cc_ref, at_ref, dv_ref,
             yf_ref, h_s, ys_s):
    lstep = pl.program_id(1)

    @pl.when(lstep == 0)
    def _():
        h_s[...] = jnp.zeros((DS, DI), F32)

    A = -jnp.exp(at_ref[...])
    iota8 = jax.lax.broadcasted_iota(jnp.int32, (8, EG), 0)
    ng = DI // EG

    def chunk_body(ci, _):
        base = pl.multiple_of(ci * 8, 8)
        bslab = [
            bb_ref[0, pl.ds(pl.multiple_of((ci * 8 + j) * DS, DS), DS), :]
            .astype(F32) for j in range(8)]
        cslab = [
            cc_ref[0, pl.ds(pl.multiple_of((ci * 8 + j) * DS, DS), DS), :]
            .astype(F32) for j in range(8)]
        for g in range(ng):
            es = g * EG
            h = h_s[:, es:es + EG]
            dtc = dt_ref[0, pl.ds(base, 8), es:es + EG]
            uc = dtc * xc_ref[0, pl.ds(base, 8), es:es + EG]
            Ag = A[:, es:es + EG]
            y8 = jnp.zeros((8, EG), F32)
            for j in range(8):
                dtj = dtc[j:j + 1, :]
                uj = uc[j:j + 1, :]
                dA = jnp.exp(Ag * dtj)
                b_bc = jnp.tile(bslab[j], (1, EG // 128))
                c_bc = jnp.tile(cslab[j], (1, EG // 128))
                h = h * dA + b_bc * uj
                yv = h * c_bc
                r = yv[:8, :] + yv[8:, :]
                r = r + pltpu.roll(r, 4, axis=0)
                r = r + pltpu.roll(r, 2, axis=0)
                r = r + pltpu.roll(r, 1, axis=0)
                y8 = jnp.where(iota8 == j, r, y8)
            ys_s[pl.ds(base, 8), es:es + EG] = y8
            h_s[:, es:es + EG] = h
        return ()

    jax.lax.fori_loop(0, LC // 8, chunk_body, ())

    yf_ref[0] = ((ys_s[...] + xc_ref[0] * dv_ref[...])
                 * zs_ref[0].astype(F32)).astype(BF16)


def _k3(dt, xc, zs, bb, cc, at, dv):
    nl = L // LC
    return pl.pallas_call(
        _k3_body,
        grid=(B_, nl),
        in_specs=[
            pl.BlockSpec((1, LC, DI), lambda b, l: (b, l, 0)),
            pl.BlockSpec((1, LC, DI), lambda b, l: (b, l, 0)),
            pl.BlockSpec((1, LC, DI), lambda b, l: (b, l, 0)),
            pl.BlockSpec((1, LC * DS, 128), lambda b, l: (b, l, 0)),
            pl.BlockSpec((1, LC * DS, 128), lambda b, l: (b, l, 0)),
            pl.BlockSpec((DS, DI), lambda b, l: (0, 0)),
            pl.BlockSpec((1, DI), lambda b, l: (0, 0)),
        ],
        out_specs=pl.BlockSpec((1, LC, DI), lambda b, l: (b, l, 0)),
        out_shape=jax.ShapeDtypeStruct((B_, L, DI), BF16),
        scratch_shapes=[
            pltpu.VMEM((DS, DI), F32),
            pltpu.VMEM((LC, DI), F32),
        ],
        compiler_params=pltpu.CompilerParams(
            dimension_semantics=("parallel", "arbitrary"),
            vmem_limit_bytes=50 * 1024 * 1024,
        ),
        name="mamba_scan",
    )(dt, xc, zs, bb, cc, at, dv)


def _k4_body(yf_ref, wot_ref, o_ref):
    o_ref[0] = jnp.dot(yf_ref[0], wot_ref[...], preferred_element_type=F32)


def _k4(yf, wot):
    nl = L // LB4
    return pl.pallas_call(
        _k4_body,
        grid=(B_, nl),
        in_specs=[
            pl.BlockSpec((1, LB4, DI), lambda b, l: (b, l, 0)),
            pl.BlockSpec((DI, DM), lambda b, l: (0, 0)),
        ],
        out_specs=pl.BlockSpec((1, LB4, DM), lambda b, l: (b, l, 0)),
        out_shape=jax.ShapeDtypeStruct((B_, L, DM), F32),
        compiler_params=pltpu.CompilerParams(
            dimension_semantics=("parallel", "arbitrary"),
        ),
        name="mamba_outproj",
    )(yf, wot)


def kernel(x, in_proj_w, conv_w, conv_b, x_proj_w, dt_proj_w, dt_proj_b,
           A_log, D, out_proj_w):
    xb = x.astype(BF16)
    wit = in_proj_w[:DI].T.astype(BF16)
    wzt = in_proj_w[DI:].T.astype(BF16)
    cwt = conv_w.T
    cb = conv_b.reshape(1, DI)
    xpwt = x_proj_w.T.astype(BF16)
    dtwt = dt_proj_w.T.astype(BF16)
    dtb = dt_proj_b.reshape(1, DI)
    at = A_log.T
    dv = D.reshape(1, DI)
    wot = out_proj_w.T.astype(BF16)

    xc, zs = _k1(xb, wit, wzt, cwt, cb)
    dt, braw, craw = _k2(xc, xpwt, dtwt, dtb)
    bbc = jnp.broadcast_to(braw.reshape(B_, L * DS, 1), (B_, L * DS, 128))
    ccc = jnp.broadcast_to(craw.reshape(B_, L * DS, 1), (B_, L * DS, 128))
    yf = _k3(dt, xc, zs, bbc, ccc, at, dv)
    return _k4(yf, wot)


# speedup vs baseline: 20.0929x; 1.1128x over previous
import jax
import jax.numpy as jnp
from jax.experimental import pallas as pl
from jax.experimental.pallas import tpu as pltpu

B_, L, DM = 2, 1024, 1024
DI, DS, DC, DR = 2048, 16, 4, 64
F32, BF16 = jnp.float32, jnp.bfloat16
LOG2E = 1.4426950408889634

E1 = 512
LC = 256
EG = 512


def _silu(x):
    return x * jax.nn.sigmoid(x)


def _softplus(x):
    return jnp.maximum(x, 0.0) + jnp.log(1.0 + jnp.exp(-jnp.abs(x)))


def _k12_body(xb_ref, wit_ref, wzt_ref, cwt_ref, cb_ref, xpwt_ref,
              dtwt_ref, dtb_ref,
              xc_ref, zs_ref, dt_ref, b_ref, c_ref, dbl_s):
    estep = pl.program_id(1)
    xb = xb_ref[0]
    xi = jnp.dot(xb, wit_ref[...], preferred_element_type=F32)
    z = jnp.dot(xb, wzt_ref[...], preferred_element_type=F32)
    acc = cwt_ref[3:4, :] * xi + cb_ref[...]
    for k in range(3):
        sh = 3 - k
        xs = jnp.concatenate(
            [jnp.zeros((sh, E1), F32), xi[:L - sh, :]], axis=0)
        acc = acc + cwt_ref[k:k + 1, :] * xs
    xc = _silu(acc)
    xc_ref[0] = xc
    zs_ref[0] = _silu(z).astype(BF16)

    part = jnp.dot(xc.astype(BF16), xpwt_ref[...],
                   preferred_element_type=F32)

    @pl.when(estep == 0)
    def _():
        dbl_s[...] = part

    @pl.when(estep > 0)
    def _():
        dbl_s[...] = dbl_s[...] + part

    @pl.when(estep == DI // E1 - 1)
    def _():
        dbl = dbl_s[...]
        dti = dbl[:, :DR].astype(BF16)
        pre = (jnp.dot(dti, dtwt_ref[...], preferred_element_type=F32)
               + dtb_ref[...])
        dt_ref[0] = _softplus(pre).astype(BF16)
        b_ref[0] = dbl[:, DR:DR + DS].astype(BF16)
        c_ref[0] = dbl[:, DR + DS:DR + 2 * DS].astype(BF16)


def _k12(xb, wit, wzt, cwt, cb, xpwt, dtwt, dtb):
    ne = DI // E1
    return pl.pallas_call(
        _k12_body,
        grid=(B_, ne),
        in_specs=[
            pl.BlockSpec((1, L, DM), lambda b, e: (b, 0, 0)),
            pl.BlockSpec((DM, E1), lambda b, e: (0, e)),
            pl.BlockSpec((DM, E1), lambda b, e: (0, e)),
            pl.BlockSpec((DC, E1), lambda b, e: (0, e)),
            pl.BlockSpec((1, E1), lambda b, e: (0, e)),
            pl.BlockSpec((E1, DR + 2 * DS), lambda b, e: (e, 0)),
            pl.BlockSpec((DR, DI), lambda b, e: (0, 0)),
            pl.BlockSpec((1, DI), lambda b, e: (0, 0)),
        ],
        out_specs=[
            pl.BlockSpec((1, L, E1), lambda b, e: (b, 0, e)),
            pl.BlockSpec((1, L, E1), lambda b, e: (b, 0, e)),
            pl.BlockSpec((1, L, DI), lambda b, e: (b, 0, 0)),
            pl.BlockSpec((1, L, DS), lambda b, e: (b, 0, 0)),
            pl.BlockSpec((1, L, DS), lambda b, e: (b, 0, 0)),
        ],
        out_shape=[
            jax.ShapeDtypeStruct((B_, L, DI), F32),
            jax.ShapeDtypeStruct((B_, L, DI), BF16),
            jax.ShapeDtypeStruct((B_, L, DI), BF16),
            jax.ShapeDtypeStruct((B_, L, DS), BF16),
            jax.ShapeDtypeStruct((B_, L, DS), BF16),
        ],
        scratch_shapes=[pltpu.VMEM((L, DR + 2 * DS), F32)],
        compiler_params=pltpu.CompilerParams(
            dimension_semantics=("parallel", "arbitrary"),
            vmem_limit_bytes=50 * 1024 * 1024,
        ),
        name="mamba_proj_conv",
    )(xb, wit, wzt, cwt, cb, xpwt, dtwt, dtb)


def _k34_body(dt_ref, xc_ref, zs_ref, bb_ref, cc_ref, at_ref, dv_ref,
              wot_ref, o_ref, h_s, ys_s):
    lstep = pl.program_id(1)

    @pl.when(lstep == 0)
    def _():
        h_s[...] = jnp.zeros((DS, DI), F32)

    A2 = -jnp.exp(at_ref[...]) * LOG2E
    iota8 = jax.lax.broadcasted_iota(jnp.int32, (8, EG), 0)
    ng = DI // EG

    def chunk_body(ci, _):
        base = pl.multiple_of(ci * 8, 8)
        bslab = [
            bb_ref[0, pl.ds(pl.multiple_of((ci * 8 + j) * DS, DS), DS), :]
            .astype(F32) for j in range(8)]
        cslab = [
            cc_ref[0, pl.ds(pl.multiple_of((ci * 8 + j) * DS, DS), DS), :]
            .astype(F32) for j in range(8)]
        for g in range(ng):
            es = g * EG
            h = h_s[:, es:es + EG]
            dtc = dt_ref[0, pl.ds(base, 8), es:es + EG].astype(F32)
            uc = dtc * xc_ref[0, pl.ds(base, 8), es:es + EG]
            Ag = A2[:, es:es + EG]
            y8 = jnp.zeros((8, EG), F32)
            for j in range(8):
                dtj = dtc[j:j + 1, :]
                uj = uc[j:j + 1, :]
                dA = jnp.exp2(Ag * dtj)
                b_bc = jnp.tile(bslab[j], (1, EG // 128))
                c_bc = jnp.tile(cslab[j], (1, EG // 128))
                h = h * dA + b_bc * uj
                yv = h * c_bc
                r = yv[:8, :] + yv[8:, :]
                r = r + pltpu.roll(r, 4, axis=0)
                r = r + pltpu.roll(r, 2, axis=0)
                r = r + pltpu.roll(r, 1, axis=0)
                y8 = jnp.where(iota8 == j, r, y8)
            ys_s[pl.ds(base, 8), es:es + EG] = y8
            h_s[:, es:es + EG] = h
        return ()

    jax.lax.fori_loop(0, LC // 8, chunk_body, ())

    yf = ((ys_s[...] + xc_ref[0] * dv_ref[...])
          * zs_ref[0].astype(F32)).astype(BF16)
    o_ref[0] = jnp.dot(yf, wot_ref[...], preferred_element_type=F32)


def _k34(dt, xc, zs, bb, cc, at, dv, wot):
    nl = L // LC
    return pl.pallas_call(
        _k34_body,
        grid=(B_, nl),
        in_specs=[
            pl.BlockSpec((1, LC, DI), lambda b, l: (b, l, 0)),
            pl.BlockSpec((1, LC, DI), lambda b, l: (b, l, 0)),
            pl.BlockSpec((1, LC, DI), lambda b, l: (b, l, 0)),
            pl.BlockSpec((1, LC * DS, 128), lambda b, l: (b, l, 0)),
            pl.BlockSpec((1, LC * DS, 128), lambda b, l: (b, l, 0)),
            pl.BlockSpec((DS, DI), lambda b, l: (0, 0)),
            pl.BlockSpec((1, DI), lambda b, l: (0, 0)),
            pl.BlockSpec((DI, DM), lambda b, l: (0, 0)),
        ],
        out_specs=pl.BlockSpec((1, LC, DM), lambda b, l: (b, l, 0)),
        out_shape=jax.ShapeDtypeStruct((B_, L, DM), F32),
        scratch_shapes=[
            pltpu.VMEM((DS, DI), F32),
            pltpu.VMEM((LC, DI), F32),
        ],
        compiler_params=pltpu.CompilerParams(
            dimension_semantics=("parallel", "arbitrary"),
            vmem_limit_bytes=50 * 1024 * 1024,
        ),
        name="mamba_scan_out",
    )(dt, xc, zs, bb, cc, at, dv, wot)


def kernel(x, in_proj_w, conv_w, conv_b, x_proj_w, dt_proj_w, dt_proj_b,
           A_log, D, out_proj_w):
    xb = x.astype(BF16)
    wit = in_proj_w[:DI].T.astype(BF16)
    wzt = in_proj_w[DI:].T.astype(BF16)
    cwt = conv_w.T
    cb = conv_b.reshape(1, DI)
    xpwt = x_proj_w.T.astype(BF16)
    dtwt = dt_proj_w.T.astype(BF16)
    dtb = dt_proj_b.reshape(1, DI)
    at = A_log.T
    dv = D.reshape(1, DI)
    wot = out_proj_w.T.astype(BF16)

    xc, zs, dt, braw, craw = _k12(xb, wit, wzt, cwt, cb, xpwt, dtwt, dtb)
    bbc = jnp.broadcast_to(braw.reshape(B_, L * DS, 1), (B_, L * DS, 128))
    ccc = jnp.broadcast_to(craw.reshape(B_, L * DS, 1), (B_, L * DS, 128))
    return _k34(dt, xc, zs, bbc, ccc, at, dv, wot)


# speedup vs baseline: 22.3834x; 1.1140x over previous
import jax
import jax.numpy as jnp
from jax.experimental import pallas as pl
from jax.experimental.pallas import tpu as pltpu

B_, L, DM = 2, 1024, 1024
DI, DS, DC, DR = 2048, 16, 4, 64
F32, BF16 = jnp.float32, jnp.bfloat16
LOG2E = 1.4426950408889634

E1 = 512
LC = 256
EG = 512


def _silu(x):
    return x * jax.nn.sigmoid(x)


def _softplus(x):
    return jnp.maximum(x, 0.0) + jnp.log(1.0 + jnp.exp(-jnp.abs(x)))


def _dot_t(a, b):
    return jax.lax.dot_general(a, b, (((1,), (1,)), ((), ())),
                               preferred_element_type=F32)


def _k12_body(x_ref, wi_ref, wz_ref, cwt_ref, cb_ref, xpw_ref,
              dtw_ref, dtb_ref,
              xc_ref, zs_ref, dt_ref, b_ref, c_ref, dbl_s):
    estep = pl.program_id(1)
    xb = x_ref[0].astype(BF16)
    xi = _dot_t(xb, wi_ref[...].astype(BF16))
    z = _dot_t(xb, wz_ref[...].astype(BF16))
    acc = cwt_ref[3:4, :] * xi + cb_ref[...]
    for k in range(3):
        sh = 3 - k
        xs = jnp.concatenate(
            [jnp.zeros((sh, E1), F32), xi[:L - sh, :]], axis=0)
        acc = acc + cwt_ref[k:k + 1, :] * xs
    xc = _silu(acc)
    xc_ref[0] = xc
    zs_ref[0] = _silu(z).astype(BF16)

    part = _dot_t(xc.astype(BF16), xpw_ref[...].astype(BF16))

    @pl.when(estep == 0)
    def _():
        dbl_s[...] = part

    @pl.when(estep > 0)
    def _():
        dbl_s[...] = dbl_s[...] + part

    @pl.when(estep == DI // E1 - 1)
    def _():
        dbl = dbl_s[...]
        dti = dbl[:, :DR].astype(BF16)
        pre = _dot_t(dti, dtw_ref[...].astype(BF16)) + dtb_ref[...]
        dt_ref[0] = _softplus(pre).astype(BF16)
        b_ref[0] = dbl[:, DR:DR + DS]
        c_ref[0] = dbl[:, DR + DS:DR + 2 * DS]


def _k12(x, in_proj_w, cwt, cb, x_proj_w, dt_proj_w, dtb):
    ne = DI // E1
    return pl.pallas_call(
        _k12_body,
        grid=(B_, ne),
        in_specs=[
            pl.BlockSpec((1, L, DM), lambda b, e: (b, 0, 0)),
            pl.BlockSpec((E1, DM), lambda b, e: (e, 0)),
            pl.BlockSpec((E1, DM), lambda b, e: (DI // E1 + e, 0)),
            pl.BlockSpec((DC, E1), lambda b, e: (0, e)),
            pl.BlockSpec((1, E1), lambda b, e: (0, e)),
            pl.BlockSpec((DR + 2 * DS, E1), lambda b, e: (0, e)),
            pl.BlockSpec((DI, DR), lambda b, e: (0, 0)),
            pl.BlockSpec((1, DI), lambda b, e: (0, 0)),
        ],
        out_specs=[
            pl.BlockSpec((1, L, E1), lambda b, e: (b, 0, e)),
            pl.BlockSpec((1, L, E1), lambda b, e: (b, 0, e)),
            pl.BlockSpec((1, L, DI), lambda b, e: (b, 0, 0)),
            pl.BlockSpec((1, L, DS), lambda b, e: (b, 0, 0)),
            pl.BlockSpec((1, L, DS), lambda b, e: (b, 0, 0)),
        ],
        out_shape=[
            jax.ShapeDtypeStruct((B_, L, DI), F32),
            jax.ShapeDtypeStruct((B_, L, DI), BF16),
            jax.ShapeDtypeStruct((B_, L, DI), BF16),
            jax.ShapeDtypeStruct((B_, L, DS), F32),
            jax.ShapeDtypeStruct((B_, L, DS), F32),
        ],
        scratch_shapes=[pltpu.VMEM((L, DR + 2 * DS), F32)],
        compiler_params=pltpu.CompilerParams(
            dimension_semantics=("parallel", "arbitrary"),
            vmem_limit_bytes=50 * 1024 * 1024,
        ),
        name="mamba_proj_conv",
    )(x, in_proj_w, in_proj_w, cwt, cb, x_proj_w, dt_proj_w, dtb)


def _k34_body(dt_ref, xc_ref, zs_ref, b_ref, c_ref, at_ref, dv_ref,
              wo_ref, o_ref, h_s, ys_s):
    lstep = pl.program_id(1)

    @pl.when(lstep == 0)
    def _():
        h_s[...] = jnp.zeros((DS, DI), F32)

    A2 = -jnp.exp(at_ref[...]) * LOG2E
    iota8 = jax.lax.broadcasted_iota(jnp.int32, (8, EG), 0)
    ng = DI // EG

    def chunk_body(ci, _):
        base = pl.multiple_of(ci * 8, 8)
        bt = jnp.transpose(b_ref[0, pl.ds(base, 8), :])
        ct = jnp.transpose(c_ref[0, pl.ds(base, 8), :])
        bslab = [jnp.broadcast_to(bt[:, j:j + 1], (DS, 128))
                 for j in range(8)]
        cslab = [jnp.broadcast_to(ct[:, j:j + 1], (DS, 128))
                 for j in range(8)]
        for g in range(ng):
            es = g * EG
            h = h_s[:, es:es + EG]
            dtc = dt_ref[0, pl.ds(base, 8), es:es + EG].astype(F32)
            uc = dtc * xc_ref[0, pl.ds(base, 8), es:es + EG]
            Ag = A2[:, es:es + EG]
            y8 = jnp.zeros((8, EG), F32)
            for j in range(8):
                dtj = dtc[j:j + 1, :]
                uj = uc[j:j + 1, :]
                dA = jnp.exp2(Ag * dtj)
                b_bc = jnp.tile(bslab[j], (1, EG // 128))
                c_bc = jnp.tile(cslab[j], (1, EG // 128))
                h = h * dA + b_bc * uj
                yv = h * c_bc
                r = yv[:8, :] + yv[8:, :]
                r = r + pltpu.roll(r, 4, axis=0)
                r = r + pltpu.roll(r, 2, axis=0)
                r = r + pltpu.roll(r, 1, axis=0)
                y8 = jnp.where(iota8 == j, r, y8)
            ys_s[pl.ds(base, 8), es:es + EG] = y8
            h_s[:, es:es + EG] = h
        return ()

    jax.lax.fori_loop(0, LC // 8, chunk_body, ())

    yf = ((ys_s[...] + xc_ref[0] * dv_ref[...])
          * zs_ref[0].astype(F32)).astype(BF16)
    o_ref[0] = _dot_t(yf, wo_ref[...].astype(BF16))


def _k34(dt, xc, zs, braw, craw, at, dv, out_proj_w):
    nl = L // LC
    return pl.pallas_call(
        _k34_body,
        grid=(B_, nl),
        in_specs=[
            pl.BlockSpec((1, LC, DI), lambda b, l: (b, l, 0)),
            pl.BlockSpec((1, LC, DI), lambda b, l: (b, l, 0)),
            pl.BlockSpec((1, LC, DI), lambda b, l: (b, l, 0)),
            pl.BlockSpec((1, LC, DS), lambda b, l: (b, l, 0)),
            pl.BlockSpec((1, LC, DS), lambda b, l: (b, l, 0)),
            pl.BlockSpec((DS, DI), lambda b, l: (0, 0)),
            pl.BlockSpec((1, DI), lambda b, l: (0, 0)),
            pl.BlockSpec((DM, DI), lambda b, l: (0, 0)),
        ],
        out_specs=pl.BlockSpec((1, LC, DM), lambda b, l: (b, l, 0)),
        out_shape=jax.ShapeDtypeStruct((B_, L, DM), F32),
        scratch_shapes=[
            pltpu.VMEM((DS, DI), F32),
            pltpu.VMEM((LC, DI), F32),
        ],
        compiler_params=pltpu.CompilerParams(
            dimension_semantics=("parallel", "arbitrary"),
            vmem_limit_bytes=50 * 1024 * 1024,
        ),
        name="mamba_scan_out",
    )(dt, xc, zs, braw, craw, at, dv, out_proj_w)


def kernel(x, in_proj_w, conv_w, conv_b, x_proj_w, dt_proj_w, dt_proj_b,
           A_log, D, out_proj_w):
    cwt = conv_w.T
    at = A_log.T
    cb = conv_b.reshape(1, DI)
    dtb = dt_proj_b.reshape(1, DI)
    dv = D.reshape(1, DI)

    xc, zs, dt, braw, craw = _k12(x, in_proj_w, cwt, cb, x_proj_w,
                                  dt_proj_w, dtb)
    return _k34(dt, xc, zs, braw, craw, at, dv, out_proj_w)


# speedup vs baseline: 22.9656x; 1.0260x over previous
import jax
import jax.numpy as jnp
from jax.experimental import pallas as pl
from jax.experimental.pallas import tpu as pltpu

B_, L, DM = 2, 1024, 1024
DI, DS, DC, DR = 2048, 16, 4, 64
F32, BF16 = jnp.float32, jnp.bfloat16
LOG2E = 1.4426950408889634

E1 = 512
LC = 256
EG = 512


def _silu(x):
    return x * jax.nn.sigmoid(x)


def _softplus(x):
    return jnp.maximum(x, 0.0) + jnp.log(1.0 + jnp.exp(-jnp.abs(x)))


def _dot_t(a, b):
    return jax.lax.dot_general(a, b, (((1,), (1,)), ((), ())),
                               preferred_element_type=F32)


def _k12_body(x_ref, wi_ref, wz_ref, cwt_ref, cb_ref, xpw_ref,
              dtw_ref, dtb_ref,
              xc_ref, zs_ref, dt_ref, b_ref, c_ref, dbl_s):
    estep = pl.program_id(1)
    xb = x_ref[0].astype(BF16)
    xi = _dot_t(xb, wi_ref[...].astype(BF16))
    z = _dot_t(xb, wz_ref[...].astype(BF16))
    acc = cwt_ref[3:4, :] * xi + cb_ref[...]
    for k in range(3):
        sh = 3 - k
        xs = jnp.concatenate(
            [jnp.zeros((sh, E1), F32), xi[:L - sh, :]], axis=0)
        acc = acc + cwt_ref[k:k + 1, :] * xs
    xc = _silu(acc)
    xc_ref[0] = xc
    zs_ref[0] = _silu(z).astype(BF16)

    part = _dot_t(xc.astype(BF16), xpw_ref[...].astype(BF16))

    @pl.when(estep == 0)
    def _():
        dbl_s[...] = part

    @pl.when(estep > 0)
    def _():
        dbl_s[...] = dbl_s[...] + part

    @pl.when(estep == DI // E1 - 1)
    def _():
        dbl = dbl_s[...]
        dti = dbl[:, :DR].astype(BF16)
        pre = _dot_t(dti, dtw_ref[...].astype(BF16)) + dtb_ref[...]
        dt_ref[0] = _softplus(pre).astype(BF16)
        b_ref[0] = dbl[:, DR:DR + DS]
        cpad = jnp.concatenate(
            [dbl[:, DR + DS:DR + 2 * DS], jnp.zeros((L, 128 - DS), F32)],
            axis=1)
        cshift = pltpu.roll(cpad, 0, axis=1, stride=DS, stride_axis=0)
        c_ref[0] = cshift.astype(BF16)


def _k12(x, in_proj_w, cwt, cb, x_proj_w, dt_proj_w, dtb):
    ne = DI // E1
    return pl.pallas_call(
        _k12_body,
        grid=(B_, ne),
        in_specs=[
            pl.BlockSpec((1, L, DM), lambda b, e: (b, 0, 0)),
            pl.BlockSpec((E1, DM), lambda b, e: (e, 0)),
            pl.BlockSpec((E1, DM), lambda b, e: (DI // E1 + e, 0)),
            pl.BlockSpec((DC, E1), lambda b, e: (0, e)),
            pl.BlockSpec((1, E1), lambda b, e: (0, e)),
            pl.BlockSpec((DR + 2 * DS, E1), lambda b, e: (0, e)),
            pl.BlockSpec((DI, DR), lambda b, e: (0, 0)),
            pl.BlockSpec((1, DI), lambda b, e: (0, 0)),
        ],
        out_specs=[
            pl.BlockSpec((1, L, E1), lambda b, e: (b, 0, e)),
            pl.BlockSpec((1, L, E1), lambda b, e: (b, 0, e)),
            pl.BlockSpec((1, L, DI), lambda b, e: (b, 0, 0)),
            pl.BlockSpec((1, L, DS), lambda b, e: (b, 0, 0)),
            pl.BlockSpec((1, L, 128), lambda b, e: (b, 0, 0)),
        ],
        out_shape=[
            jax.ShapeDtypeStruct((B_, L, DI), F32),
            jax.ShapeDtypeStruct((B_, L, DI), BF16),
            jax.ShapeDtypeStruct((B_, L, DI), BF16),
            jax.ShapeDtypeStruct((B_, L, DS), F32),
            jax.ShapeDtypeStruct((B_, L, 128), BF16),
        ],
        scratch_shapes=[pltpu.VMEM((L, DR + 2 * DS), F32)],
        compiler_params=pltpu.CompilerParams(
            dimension_semantics=("parallel", "arbitrary"),
            vmem_limit_bytes=50 * 1024 * 1024,
        ),
        name="mamba_proj_conv",
    )(x, in_proj_w, in_proj_w, cwt, cb, x_proj_w, dt_proj_w, dtb)


def _k34_body(dt_ref, xc_ref, zs_ref, b_ref, c_ref, at_ref, dv_ref,
              wo_ref, o_ref, h_s, ys_s, hst_s):
    lstep = pl.program_id(1)

    @pl.when(lstep == 0)
    def _():
        h_s[...] = jnp.zeros((DS, DI), F32)

    A2 = -jnp.exp(at_ref[...]) * LOG2E
    ng = DI // EG

    def chunk_body(ci, _):
        base = pl.multiple_of(ci * 8, 8)
        bt = jnp.transpose(b_ref[0, pl.ds(base, 8), :])
        bslab = [jnp.broadcast_to(bt[:, j:j + 1], (DS, 128))
                 for j in range(8)]
        cm8 = c_ref[0, pl.ds(base, 8), :]
        for g in range(ng):
            es = g * EG
            h = h_s[:, es:es + EG]
            dtc = dt_ref[0, pl.ds(base, 8), es:es + EG].astype(F32)
            uc = dtc * xc_ref[0, pl.ds(base, 8), es:es + EG]
            Ag = A2[:, es:es + EG]
            for j in range(8):
                dtj = dtc[j:j + 1, :]
                uj = uc[j:j + 1, :]
                dA = jnp.exp2(Ag * dtj)
                b_bc = jnp.tile(bslab[j], (1, EG // 128))
                h = h * dA + b_bc * uj
                hst_s[DS * j:DS * (j + 1), es:es + EG] = h.astype(BF16)
            ys_s[pl.ds(base, 8), es:es + EG] = jnp.dot(
                cm8, hst_s[:, es:es + EG], preferred_element_type=F32)
            h_s[:, es:es + EG] = h
        return ()

    jax.lax.fori_loop(0, LC // 8, chunk_body, ())

    yf = ((ys_s[...] + xc_ref[0] * dv_ref[...])
          * zs_ref[0].astype(F32)).astype(BF16)
    o_ref[0] = _dot_t(yf, wo_ref[...].astype(BF16))


def _k34(dt, xc, zs, braw, craw, at, dv, out_proj_w):
    nl = L // LC
    return pl.pallas_call(
        _k34_body,
        grid=(B_, nl),
        in_specs=[
            pl.BlockSpec((1, LC, DI), lambda b, l: (b, l, 0)),
            pl.BlockSpec((1, LC, DI), lambda b, l: (b, l, 0)),
            pl.BlockSpec((1, LC, DI), lambda b, l: (b, l, 0)),
            pl.BlockSpec((1, LC, DS), lambda b, l: (b, l, 0)),
            pl.BlockSpec((1, LC, 128), lambda b, l: (b, l, 0)),
            pl.BlockSpec((DS, DI), lambda b, l: (0, 0)),
            pl.BlockSpec((1, DI), lambda b, l: (0, 0)),
            pl.BlockSpec((DM, DI), lambda b, l: (0, 0)),
        ],
        out_specs=pl.BlockSpec((1, LC, DM), lambda b, l: (b, l, 0)),
        out_shape=jax.ShapeDtypeStruct((B_, L, DM), F32),
        scratch_shapes=[
            pltpu.VMEM((DS, DI), F32),
            pltpu.VMEM((LC, DI), F32),
            pltpu.VMEM((DS * 8, DI), BF16),
        ],
        compiler_params=pltpu.CompilerParams(
            dimension_semantics=("parallel", "arbitrary"),
            vmem_limit_bytes=50 * 1024 * 1024,
        ),
        name="mamba_scan_out",
    )(dt, xc, zs, braw, craw, at, dv, out_proj_w)


def kernel(x, in_proj_w, conv_w, conv_b, x_proj_w, dt_proj_w, dt_proj_b,
           A_log, D, out_proj_w):
    cwt = conv_w.T
    at = A_log.T
    cb = conv_b.reshape(1, DI)
    dtb = dt_proj_b.reshape(1, DI)
    dv = D.reshape(1, DI)

    xc, zs, dt, braw, craw = _k12(x, in_proj_w, cwt, cb, x_proj_w,
                                  dt_proj_w, dtb)
    return _k34(dt, xc, zs, braw, craw, at, dv, out_proj_w)
